# baseline (speedup 1.0000x reference)
import functools

import jax
import jax.numpy as jnp
from jax import lax
from jax.experimental import pallas as pl
from jax.experimental.pallas import tpu as pltpu


def _threefry_bits(j):
    ks0 = jnp.uint32(0)
    ks1 = jnp.uint32(42)
    ks2 = jnp.uint32(0x1BD11BDA ^ 42)

    x0 = jnp.zeros_like(j) + ks0
    x1 = j + ks1

    rots = ((13, 15, 26, 6), (17, 29, 16, 24))
    ks = (ks0, ks1, ks2)
    for i in range(5):
        for r in rots[i % 2]:
            x0 = x0 + x1
            x1 = (x1 << r) | (x1 >> (32 - r))
            x1 = x1 ^ x0
        x0 = x0 + ks[(i + 1) % 3]
        x1 = x1 + ks[(i + 2) % 3] + jnp.uint32(i + 1)
    return x0 ^ x1


def _body(vocab_size, n_steps, chunk, logits_ref, invt_ref, out_ref,
          best_val, best_idx):
    g = pl.program_id(0)
    b = logits_ref.shape[0]

    x = logits_ref[...]
    col = lax.broadcasted_iota(jnp.int32, (b, chunk), 1) + g * chunk
    row = lax.broadcasted_iota(jnp.int32, (b, chunk), 0)
    j = (row * vocab_size + col).astype(jnp.uint32)

    bits = _threefry_bits(j)
    fb = (bits >> jnp.uint32(9)) | jnp.uint32(0x3F800000)
    u = lax.bitcast_convert_type(fb, jnp.float32) - jnp.float32(1.0)
    q = -jnp.log1p(-u)

    s = x * invt_ref[...] - jnp.log(q)
    valid = col < vocab_size
    s = jnp.where(valid, s, -jnp.inf)

    m = jnp.max(s, axis=1, keepdims=True)
    idx = jnp.min(jnp.where(s == m, col, vocab_size), axis=1, keepdims=True)

    @pl.when(g == 0)
    def _init():
        best_val[...] = jnp.full_like(best_val, -jnp.inf)
        best_idx[...] = jnp.zeros_like(best_idx)

    better = m > best_val[...]
    best_idx[...] = jnp.where(better, idx, best_idx[...])
    best_val[...] = jnp.where(better, m, best_val[...])

    @pl.when(g == n_steps - 1)
    def _done():
        out_ref[...] = best_idx[...]


def kernel(logits, temperatures):
    b, v = logits.shape
    chunk = 16384
    n_steps = pl.cdiv(v, chunk)
    invt = (1.0 / temperatures.astype(jnp.float32)).reshape(b, 1)

    body = functools.partial(_body, v, n_steps, chunk)

    out = pl.pallas_call(
        body,
        grid=(n_steps,),
        in_specs=[
            pl.BlockSpec((b, chunk), lambda g: (0, g)),
            pl.BlockSpec((b, 1), lambda g: (0, 0)),
        ],
        out_specs=pl.BlockSpec((b, 1), lambda g: (0, 0)),
        out_shape=jax.ShapeDtypeStruct((b, 1), jnp.int32),
        scratch_shapes=[
            pltpu.VMEM((b, 1), jnp.float32),
            pltpu.VMEM((b, 1), jnp.int32),
        ],
    )(logits.astype(jnp.float32), invt)
    return out.reshape(b)
